# baseline (speedup 1.0000x reference)
import jax
import jax.numpy as jnp
from jax.experimental import pallas as pl

NUM_EXPERTS = 64
D_MODEL = 2048
BLOCK_T = 512


def _router_body(x_ref, wt_ref, oh_ref, mp_ref, lg_ref):
    x = x_ref[...]
    wt = wt_ref[...]
    logits = jax.lax.dot_general(
        x, wt, (((1,), (0,)), ((), ())),
        preferred_element_type=jnp.float32)
    m = jnp.max(logits, axis=-1, keepdims=True)
    e = jnp.exp(logits - m)
    s = jnp.sum(e, axis=-1, keepdims=True)
    probs = e / s
    mp = jnp.max(probs, axis=-1, keepdims=True)
    lane = jax.lax.broadcasted_iota(jnp.int32, probs.shape, 1)
    idx = jnp.min(jnp.where(probs == mp, lane, NUM_EXPERTS),
                  axis=-1, keepdims=True)
    oh_ref[...] = (lane == idx).astype(jnp.int32)
    mp_ref[...] = mp
    lg_ref[...] = logits


def kernel(x, W):
    n = x.shape[0]
    wt = W.T
    one_hot, max_probs, logits = pl.pallas_call(
        _router_body,
        grid=(n // BLOCK_T,),
        in_specs=[
            pl.BlockSpec((BLOCK_T, D_MODEL), lambda i: (i, 0)),
            pl.BlockSpec((D_MODEL, NUM_EXPERTS), lambda i: (0, 0)),
        ],
        out_specs=[
            pl.BlockSpec((BLOCK_T, NUM_EXPERTS), lambda i: (i, 0)),
            pl.BlockSpec((BLOCK_T, 1), lambda i: (i, 0)),
            pl.BlockSpec((BLOCK_T, NUM_EXPERTS), lambda i: (i, 0)),
        ],
        out_shape=[
            jax.ShapeDtypeStruct((n, NUM_EXPERTS), jnp.int32),
            jax.ShapeDtypeStruct((n, 1), jnp.float32),
            jax.ShapeDtypeStruct((n, NUM_EXPERTS), jnp.float32),
        ],
    )(x, wt)
    return one_hot, max_probs, logits


# speedup vs baseline: 1.1140x; 1.1140x over previous
import jax
import jax.numpy as jnp
from jax.experimental import pallas as pl

NUM_EXPERTS = 64
D_MODEL = 2048
BLOCK_T = 2048


def _router_body(x_ref, wt_ref, oh_ref, mp_ref, lg_ref):
    x = x_ref[...]
    wt = wt_ref[...]
    logits = jax.lax.dot_general(
        x, wt, (((1,), (0,)), ((), ())),
        preferred_element_type=jnp.float32)
    m = jnp.max(logits, axis=-1, keepdims=True)
    e = jnp.exp(logits - m)
    s = jnp.sum(e, axis=-1, keepdims=True)
    probs = e / s
    mp = jnp.max(probs, axis=-1, keepdims=True)
    lane = jax.lax.broadcasted_iota(jnp.int32, probs.shape, 1)
    idx = jnp.min(jnp.where(probs == mp, lane, NUM_EXPERTS),
                  axis=-1, keepdims=True)
    oh_ref[...] = (lane == idx).astype(jnp.int32)
    mp_ref[...] = mp
    lg_ref[...] = logits


def kernel(x, W):
    n = x.shape[0]
    wt = W.T
    one_hot, max_probs, logits = pl.pallas_call(
        _router_body,
        grid=(n // BLOCK_T,),
        in_specs=[
            pl.BlockSpec((BLOCK_T, D_MODEL), lambda i: (i, 0)),
            pl.BlockSpec((D_MODEL, NUM_EXPERTS), lambda i: (0, 0)),
        ],
        out_specs=[
            pl.BlockSpec((BLOCK_T, NUM_EXPERTS), lambda i: (i, 0)),
            pl.BlockSpec((BLOCK_T, 1), lambda i: (i, 0)),
            pl.BlockSpec((BLOCK_T, NUM_EXPERTS), lambda i: (i, 0)),
        ],
        out_shape=[
            jax.ShapeDtypeStruct((n, NUM_EXPERTS), jnp.int32),
            jax.ShapeDtypeStruct((n, 1), jnp.float32),
            jax.ShapeDtypeStruct((n, NUM_EXPERTS), jnp.float32),
        ],
    )(x, wt)
    return one_hot, max_probs, logits


# speedup vs baseline: 1.1250x; 1.0099x over previous
import jax
import jax.numpy as jnp
from jax.experimental import pallas as pl

NUM_EXPERTS = 64
D_MODEL = 2048
BLOCK_T = 2048


def _router_body(x0_ref, x1_ref, x2_ref, x3_ref, wt_ref, oh_ref, mp_ref, lg_ref):
    wt = wt_ref[...]
    ck = D_MODEL // 4
    logits = None
    for j, xr in enumerate((x0_ref, x1_ref, x2_ref, x3_ref)):
        part = jax.lax.dot_general(
            xr[...], wt[j * ck:(j + 1) * ck, :], (((1,), (0,)), ((), ())),
            preferred_element_type=jnp.float32)
        logits = part if logits is None else logits + part
    m = jnp.max(logits, axis=-1, keepdims=True)
    e = jnp.exp(logits - m)
    s = jnp.sum(e, axis=-1, keepdims=True)
    probs = e / s
    mp = jnp.max(probs, axis=-1, keepdims=True)
    lane = jax.lax.broadcasted_iota(jnp.int32, probs.shape, 1)
    idx = jnp.min(jnp.where(probs == mp, lane, NUM_EXPERTS),
                  axis=-1, keepdims=True)
    oh_ref[...] = (lane == idx).astype(jnp.int32)
    mp_ref[...] = mp
    lg_ref[...] = logits


def kernel(x, W):
    n = x.shape[0]
    wt = W.T
    one_hot, max_probs, logits = pl.pallas_call(
        _router_body,
        grid=(n // BLOCK_T,),
        in_specs=[
            pl.BlockSpec((BLOCK_T, D_MODEL // 4), lambda i: (i, 0)),
            pl.BlockSpec((BLOCK_T, D_MODEL // 4), lambda i: (i, 1)),
            pl.BlockSpec((BLOCK_T, D_MODEL // 4), lambda i: (i, 2)),
            pl.BlockSpec((BLOCK_T, D_MODEL // 4), lambda i: (i, 3)),
            pl.BlockSpec((D_MODEL, NUM_EXPERTS), lambda i: (0, 0)),
        ],
        out_specs=[
            pl.BlockSpec((BLOCK_T, NUM_EXPERTS), lambda i: (i, 0)),
            pl.BlockSpec((BLOCK_T, 1), lambda i: (i, 0)),
            pl.BlockSpec((BLOCK_T, NUM_EXPERTS), lambda i: (i, 0)),
        ],
        out_shape=[
            jax.ShapeDtypeStruct((n, NUM_EXPERTS), jnp.int32),
            jax.ShapeDtypeStruct((n, 1), jnp.float32),
            jax.ShapeDtypeStruct((n, NUM_EXPERTS), jnp.float32),
        ],
    )(x, x, x, x, wt)
    return one_hot, max_probs, logits


# speedup vs baseline: 1.7750x; 1.5778x over previous
import jax
import jax.numpy as jnp
from jax.experimental import pallas as pl

NUM_EXPERTS = 64
D_MODEL = 2048
BLOCK_T = 2048


def _router_body(x_ref, w_ref, oh_ref, mp_ref, lg_ref):
    x = x_ref[...]
    w = w_ref[...]
    logits = jax.lax.dot_general(
        w, x, (((1,), (1,)), ((), ())),
        preferred_element_type=jnp.float32)
    m = jnp.max(logits, axis=0, keepdims=True)
    e = jnp.exp(logits - m)
    s = jnp.sum(e, axis=0, keepdims=True)
    probs = e / s
    mp = jnp.max(probs, axis=0, keepdims=True)
    row = jax.lax.broadcasted_iota(jnp.int32, probs.shape, 0)
    idx = jnp.min(jnp.where(probs == mp, row, NUM_EXPERTS),
                  axis=0, keepdims=True)
    oh_ref[...] = (row == idx).astype(jnp.int32)
    mp_ref[...] = mp
    lg_ref[...] = logits


def kernel(x, W):
    n = x.shape[0]
    oh_t, mp_t, lg_t = pl.pallas_call(
        _router_body,
        grid=(n // BLOCK_T,),
        in_specs=[
            pl.BlockSpec((BLOCK_T, D_MODEL), lambda i: (i, 0)),
            pl.BlockSpec((NUM_EXPERTS, D_MODEL), lambda i: (0, 0)),
        ],
        out_specs=[
            pl.BlockSpec((NUM_EXPERTS, BLOCK_T), lambda i: (0, i)),
            pl.BlockSpec((1, BLOCK_T), lambda i: (0, i)),
            pl.BlockSpec((NUM_EXPERTS, BLOCK_T), lambda i: (0, i)),
        ],
        out_shape=[
            jax.ShapeDtypeStruct((NUM_EXPERTS, n), jnp.int32),
            jax.ShapeDtypeStruct((1, n), jnp.float32),
            jax.ShapeDtypeStruct((NUM_EXPERTS, n), jnp.float32),
        ],
    )(x, W)
    return oh_t.T, mp_t.T, lg_t.T


# speedup vs baseline: 1.7842x; 1.0052x over previous
import jax
import jax.numpy as jnp
from jax.experimental import pallas as pl

NUM_EXPERTS = 64
D_MODEL = 2048
BLOCK_T = 1024


def _router_body(x_ref, w_ref, oh_ref, mp_ref, lg_ref):
    x = x_ref[...]
    w = w_ref[...]
    logits = jax.lax.dot_general(
        w, x, (((1,), (1,)), ((), ())),
        preferred_element_type=jnp.float32)
    m = jnp.max(logits, axis=0, keepdims=True)
    e = jnp.exp(logits - m)
    s = jnp.sum(e, axis=0, keepdims=True)
    probs = e / s
    mp = jnp.max(probs, axis=0, keepdims=True)
    row = jax.lax.broadcasted_iota(jnp.int32, probs.shape, 0)
    idx = jnp.min(jnp.where(probs == mp, row, NUM_EXPERTS),
                  axis=0, keepdims=True)
    oh_ref[...] = (row == idx).astype(jnp.int32)
    mp_ref[...] = mp
    lg_ref[...] = logits


def kernel(x, W):
    n = x.shape[0]
    oh_t, mp_t, lg_t = pl.pallas_call(
        _router_body,
        grid=(n // BLOCK_T,),
        in_specs=[
            pl.BlockSpec((BLOCK_T, D_MODEL), lambda i: (i, 0)),
            pl.BlockSpec((NUM_EXPERTS, D_MODEL), lambda i: (0, 0)),
        ],
        out_specs=[
            pl.BlockSpec((NUM_EXPERTS, BLOCK_T), lambda i: (0, i)),
            pl.BlockSpec((1, BLOCK_T), lambda i: (0, i)),
            pl.BlockSpec((NUM_EXPERTS, BLOCK_T), lambda i: (0, i)),
        ],
        out_shape=[
            jax.ShapeDtypeStruct((NUM_EXPERTS, n), jnp.int32),
            jax.ShapeDtypeStruct((1, n), jnp.float32),
            jax.ShapeDtypeStruct((NUM_EXPERTS, n), jnp.float32),
        ],
    )(x, W)
    return oh_t.T, mp_t.T, lg_t.T
